# baseline (device time: 148750 ns/iter reference)
import functools

import jax
import jax.numpy as jnp
from jax import lax
from jax.experimental import pallas as pl
from jax.experimental.pallas import tpu as pltpu

N_DEV = 4
SQ = 2048
SKV_LOC = 2048
D = 1024
H = 8
DH = 128
QT = 512
N_QT = SQ // QT
KC = 512
BLK = 64
HT = QT // 2
SCALE = 0.08838834764831843
N_SEM = N_QT + 1

_MESH = pl.DeviceIdType.MESH


def _rows_for(slot):
    if slot < N_QT - 1:
        return pl.ds(slot * QT, QT)
    if slot == N_QT - 1:
        return pl.ds((N_QT - 1) * QT, HT)
    return pl.ds((N_QT - 1) * QT + HT, HT)


def _body(x_ref, wq_ref, k_ref, v_ref, wo_ref, out_ref,
          ctx_ref, qtile_ref, oacc_ref, lacc_ref,
          ssemA, ssemB, rsem):
    my = lax.axis_index("i")
    left = lax.rem(my + N_DEV - 1, N_DEV)
    right = lax.rem(my + 1, N_DEV)

    barrier = pltpu.get_barrier_semaphore()
    for nbr in (left, right):
        pl.semaphore_signal(barrier, inc=1, device_id=(nbr,),
                            device_id_type=_MESH)
    pl.semaphore_wait(barrier, 2)

    def seg_rdma(slot, send_sems, recv_slot, target):
        return pltpu.make_async_remote_copy(
            src_ref=ctx_ref.at[_rows_for(slot)],
            dst_ref=ctx_ref.at[_rows_for(recv_slot)],
            send_sem=send_sems.at[slot], recv_sem=rsem.at[recv_slot],
            device_id=(target,), device_id_type=_MESH)

    def project_out(slot):
        rows = _rows_for(slot)
        out_ref[rows, :] = jnp.dot(ctx_ref[rows, :], wo_ref[...],
                                   preferred_element_type=jnp.float32)

    @pl.when(my == 0)
    def _producer():
        sends = []
        for qt in range(N_QT):
            rows = pl.ds(qt * QT, QT)
            qtile_ref[...] = jnp.dot(x_ref[rows, :], wq_ref[...],
                                     preferred_element_type=jnp.float32)
            oacc_ref[...] = jnp.zeros_like(oacc_ref)
            lacc_ref[...] = jnp.zeros_like(lacc_ref)
            for kc in range(qt + 1):
                kr = pl.ds(kc * KC, KC)
                for h in range(H):
                    hs = slice(h * DH, (h + 1) * DH)
                    s = lax.dot_general(
                        qtile_ref[:, hs], k_ref[kr, hs],
                        (((1,), (1,)), ((), ())),
                        preferred_element_type=jnp.float32) * SCALE
                    if kc == qt:
                        i_blk = lax.broadcasted_iota(jnp.int32, (QT, KC), 0) // BLK
                        j_blk = lax.broadcasted_iota(jnp.int32, (QT, KC), 1) // BLK
                        w = jnp.where(j_blk <= i_blk, jnp.exp(s), 0.0)
                    else:
                        w = jnp.exp(s)
                    oacc_ref[:, hs] = oacc_ref[:, hs] + lax.dot_general(
                        w, v_ref[kr, hs], (((1,), (0,)), ((), ())),
                        preferred_element_type=jnp.float32)
                    lacc_ref[:, h:h + 1] = (lacc_ref[:, h:h + 1]
                                            + jnp.sum(w, axis=1, keepdims=True))
            for h in range(H):
                hs = slice(h * DH, (h + 1) * DH)
                ctx_ref[rows, hs] = oacc_ref[:, hs] / lacc_ref[:, h:h + 1]
            if qt < N_QT - 1:
                plan = (((qt,), ssemA, 1), ((qt,), ssemB, 3))
            else:
                plan = (((3, 4), ssemA, 1), ((4, 3), ssemB, 3))
            for slots, send_sems, target in plan:
                for slot in slots:
                    rdma = seg_rdma(slot, send_sems, slot, target)
                    rdma.start()
                    sends.append(rdma)
        for slot in range(N_SEM):
            project_out(slot)
        for rdma in sends:
            rdma.wait_send()

    @pl.when(my == 1)
    def _forwarder():
        fwds = []
        for slot in range(N_QT):
            seg_rdma(slot, ssemA, slot, 0).wait_recv()
            fwd = seg_rdma(slot, ssemB, slot, 2)
            fwd.start()
            fwds.append(fwd)
            project_out(slot)
        seg_rdma(4, ssemA, 4, 0).wait_recv()
        project_out(4)
        for fwd in fwds:
            fwd.wait_send()

    @pl.when(my == 3)
    def _forwarder_b():
        seg_rdma(4, ssemA, 4, 0).wait_recv()
        fwd = seg_rdma(4, ssemB, 4, 2)
        fwd.start()
        project_out(4)
        for slot in range(N_QT):
            seg_rdma(slot, ssemA, slot, 0).wait_recv()
            project_out(slot)
        fwd.wait_send()

    @pl.when(my == 2)
    def _receiver():
        for slot in range(N_SEM):
            seg_rdma(slot, ssemA, slot, 0).wait_recv()
            project_out(slot)

    @functools.partial(pl.run_scoped, exit_sem=pltpu.SemaphoreType.REGULAR)
    def _(exit_sem):
        for nbr in (left, right):
            pl.semaphore_signal(exit_sem, inc=1, device_id=(nbr,),
                                device_id_type=_MESH)
        pl.semaphore_wait(exit_sem, 2)


def kernel(x, Wq, K_ext, V_ext, Wo):
    x2 = x.reshape(SQ, D)
    k2 = K_ext.reshape(SKV_LOC, H * DH)
    v2 = V_ext.reshape(SKV_LOC, H * DH)
    out = pl.pallas_call(
        _body,
        out_shape=jax.ShapeDtypeStruct((SQ, D), jnp.float32),
        in_specs=[pl.BlockSpec(memory_space=pltpu.VMEM)] * 5,
        out_specs=pl.BlockSpec(memory_space=pltpu.VMEM),
        scratch_shapes=[
            pltpu.VMEM((SQ, D), jnp.float32),
            pltpu.VMEM((QT, D), jnp.float32),
            pltpu.VMEM((QT, D), jnp.float32),
            pltpu.VMEM((QT, 128), jnp.float32),
            pltpu.SemaphoreType.DMA((N_SEM,)),
            pltpu.SemaphoreType.DMA((N_SEM,)),
            pltpu.SemaphoreType.DMA((N_SEM,)),
        ],
        compiler_params=pltpu.CompilerParams(
            collective_id=0,
            vmem_limit_bytes=100 * 1024 * 1024,
        ),
    )(x2, Wq, k2, v2, Wo)
    return out.reshape(1, SQ, D)


# device time: 98055 ns/iter; 1.5170x vs baseline; 1.5170x over previous
import functools

import jax
import jax.numpy as jnp
from jax import lax
from jax.experimental import pallas as pl
from jax.experimental.pallas import tpu as pltpu

N_DEV = 4
SQ = 2048
SKV_LOC = 2048
D = 1024
H = 8
DH = 128
QT = 512
N_QT = SQ // QT
KC = 512
BLK = 64
HT = QT // 2
SCALE = 0.08838834764831843
N_SEM = N_QT + 1

_MESH = pl.DeviceIdType.MESH


def _rows_for(slot):
    if slot < N_QT - 1:
        return pl.ds(slot * QT, QT)
    if slot == N_QT - 1:
        return pl.ds((N_QT - 1) * QT, HT)
    return pl.ds((N_QT - 1) * QT + HT, HT)


def _body(x_ref, wq_ref, k_ref, v_ref, wo_ref, out_ref,
          ctx_ref, qtile_ref, oacc_ref, lacc_ref,
          ssemA, ssemB, rsem):
    my = lax.axis_index("i")
    left = lax.rem(my + N_DEV - 1, N_DEV)
    right = lax.rem(my + 1, N_DEV)

    barrier = pltpu.get_barrier_semaphore()
    for nbr in (left, right):
        pl.semaphore_signal(barrier, inc=1, device_id=(nbr,),
                            device_id_type=_MESH)
    pl.semaphore_wait(barrier, 2)

    def seg_rdma(slot, send_sems, recv_slot, target):
        return pltpu.make_async_remote_copy(
            src_ref=ctx_ref.at[_rows_for(slot)],
            dst_ref=ctx_ref.at[_rows_for(recv_slot)],
            send_sem=send_sems.at[slot], recv_sem=rsem.at[recv_slot],
            device_id=(target,), device_id_type=_MESH)

    def project_out(slot):
        rows = _rows_for(slot)
        out_ref[rows, :] = jnp.dot(ctx_ref[rows, :].astype(jnp.float32),
                                   wo_ref[...],
                                   preferred_element_type=jnp.float32)

    @pl.when(my == 0)
    def _producer():
        sends = []
        for qt in range(N_QT):
            rows = pl.ds(qt * QT, QT)
            qtile_ref[...] = jnp.dot(x_ref[rows, :], wq_ref[...],
                                     preferred_element_type=jnp.float32)
            oacc_ref[...] = jnp.zeros_like(oacc_ref)
            lacc_ref[...] = jnp.zeros_like(lacc_ref)
            for kc in range(qt + 1):
                kr = pl.ds(kc * KC, KC)
                for h in range(H):
                    hs = slice(h * DH, (h + 1) * DH)
                    s = lax.dot_general(
                        qtile_ref[:, hs], k_ref[kr, hs],
                        (((1,), (1,)), ((), ())),
                        preferred_element_type=jnp.float32) * SCALE
                    if kc == qt:
                        i_blk = lax.broadcasted_iota(jnp.int32, (QT, KC), 0) // BLK
                        j_blk = lax.broadcasted_iota(jnp.int32, (QT, KC), 1) // BLK
                        w = jnp.where(j_blk <= i_blk, jnp.exp(s), 0.0)
                    else:
                        w = jnp.exp(s)
                    oacc_ref[:, hs] = oacc_ref[:, hs] + lax.dot_general(
                        w, v_ref[kr, hs], (((1,), (0,)), ((), ())),
                        preferred_element_type=jnp.float32)
                    lacc_ref[:, h:h + 1] = (lacc_ref[:, h:h + 1]
                                            + jnp.sum(w, axis=1, keepdims=True))
            for h in range(H):
                hs = slice(h * DH, (h + 1) * DH)
                ctx_ref[rows, hs] = (oacc_ref[:, hs]
                                     / lacc_ref[:, h:h + 1]
                                     ).astype(jnp.bfloat16)
            if qt < N_QT - 1:
                plan = (((qt,), ssemA, 1), ((qt,), ssemB, 3))
            else:
                plan = (((3, 4), ssemA, 1), ((4, 3), ssemB, 3))
            for slots, send_sems, target in plan:
                for slot in slots:
                    rdma = seg_rdma(slot, send_sems, slot, target)
                    rdma.start()
                    sends.append(rdma)
        for slot in range(N_SEM):
            project_out(slot)
        for rdma in sends:
            rdma.wait_send()

    @pl.when(my == 1)
    def _forwarder():
        fwds = []
        for slot in range(N_QT):
            seg_rdma(slot, ssemA, slot, 0).wait_recv()
            fwd = seg_rdma(slot, ssemB, slot, 2)
            fwd.start()
            fwds.append(fwd)
            project_out(slot)
        seg_rdma(4, ssemA, 4, 0).wait_recv()
        project_out(4)
        for fwd in fwds:
            fwd.wait_send()

    @pl.when(my == 3)
    def _forwarder_b():
        seg_rdma(4, ssemA, 4, 0).wait_recv()
        fwd = seg_rdma(4, ssemB, 4, 2)
        fwd.start()
        project_out(4)
        for slot in range(N_QT):
            seg_rdma(slot, ssemA, slot, 0).wait_recv()
            project_out(slot)
        fwd.wait_send()

    @pl.when(my == 2)
    def _receiver():
        for slot in range(N_SEM):
            seg_rdma(slot, ssemA, slot, 0).wait_recv()
            project_out(slot)

    @functools.partial(pl.run_scoped, exit_sem=pltpu.SemaphoreType.REGULAR)
    def _(exit_sem):
        for nbr in (left, right):
            pl.semaphore_signal(exit_sem, inc=1, device_id=(nbr,),
                                device_id_type=_MESH)
        pl.semaphore_wait(exit_sem, 2)


def kernel(x, Wq, K_ext, V_ext, Wo):
    x2 = x.reshape(SQ, D)
    k2 = K_ext.reshape(SKV_LOC, H * DH)
    v2 = V_ext.reshape(SKV_LOC, H * DH)
    out = pl.pallas_call(
        _body,
        out_shape=jax.ShapeDtypeStruct((SQ, D), jnp.float32),
        in_specs=[pl.BlockSpec(memory_space=pltpu.VMEM)] * 5,
        out_specs=pl.BlockSpec(memory_space=pltpu.VMEM),
        scratch_shapes=[
            pltpu.VMEM((SQ, D), jnp.bfloat16),
            pltpu.VMEM((QT, D), jnp.float32),
            pltpu.VMEM((QT, D), jnp.float32),
            pltpu.VMEM((QT, 128), jnp.float32),
            pltpu.SemaphoreType.DMA((N_SEM,)),
            pltpu.SemaphoreType.DMA((N_SEM,)),
            pltpu.SemaphoreType.DMA((N_SEM,)),
        ],
        compiler_params=pltpu.CompilerParams(
            collective_id=0,
            vmem_limit_bytes=100 * 1024 * 1024,
        ),
    )(x2, Wq, k2, v2, Wo)
    return out.reshape(1, SQ, D)


# device time: 97506 ns/iter; 1.5255x vs baseline; 1.0056x over previous
import functools

import jax
import jax.numpy as jnp
from jax import lax
from jax.experimental import pallas as pl
from jax.experimental.pallas import tpu as pltpu

N_DEV = 4
SQ = 2048
SKV_LOC = 2048
D = 1024
H = 8
DH = 128
QT = 512
N_QT = SQ // QT
KC = 512
BLK = 64
HT = QT // 2
SCALE = 0.08838834764831843
N_SEM = N_QT + 1

_MESH = pl.DeviceIdType.MESH


def _rows_for(slot):
    if slot < N_QT - 1:
        return pl.ds(slot * QT, QT)
    if slot == N_QT - 1:
        return pl.ds((N_QT - 1) * QT, HT)
    return pl.ds((N_QT - 1) * QT + HT, HT)


def _body(x_hbm_ref, wq_ref, k_ref, v_ref, wo_ref, out_ref,
          ctx_ref, xbuf_ref, qbf_ref, kbf_ref, vbf_ref,
          oacc_ref, lacc_ref,
          xsem, ssemA, ssemB, rsem):
    my = lax.axis_index("i")
    left = lax.rem(my + N_DEV - 1, N_DEV)
    right = lax.rem(my + 1, N_DEV)

    barrier = pltpu.get_barrier_semaphore()
    for nbr in (left, right):
        pl.semaphore_signal(barrier, inc=1, device_id=(nbr,),
                            device_id_type=_MESH)
    pl.semaphore_wait(barrier, 2)

    def seg_rdma(slot, send_sems, recv_slot, target):
        return pltpu.make_async_remote_copy(
            src_ref=ctx_ref.at[_rows_for(slot)],
            dst_ref=ctx_ref.at[_rows_for(recv_slot)],
            send_sem=send_sems.at[slot], recv_sem=rsem.at[recv_slot],
            device_id=(target,), device_id_type=_MESH)

    def project_out(slot):
        rows = _rows_for(slot)
        out_ref[rows, :] = jnp.dot(ctx_ref[rows, :].astype(jnp.float32),
                                   wo_ref[...],
                                   preferred_element_type=jnp.float32)

    def x_copy(qt, slot):
        return pltpu.make_async_copy(
            x_hbm_ref.at[pl.ds(qt * QT, QT)], xbuf_ref.at[slot],
            xsem.at[slot])

    @pl.when(my == 0)
    def _producer():
        sends = []
        x_copy(0, 0).start()
        for qt in range(N_QT):
            rows = pl.ds(qt * QT, QT)
            x_copy(qt, qt % 2).wait()
            if qt + 1 < N_QT:
                x_copy(qt + 1, (qt + 1) % 2).start()
            qbf_ref[...] = jnp.dot(
                xbuf_ref[qt % 2], wq_ref[...],
                preferred_element_type=jnp.float32).astype(jnp.bfloat16)
            kr_new = pl.ds(qt * KC, KC)
            kbf_ref[kr_new, :] = k_ref[kr_new, :].astype(jnp.bfloat16)
            vbf_ref[kr_new, :] = v_ref[kr_new, :].astype(jnp.bfloat16)
            oacc_ref[...] = jnp.zeros_like(oacc_ref)
            lacc_ref[...] = jnp.zeros_like(lacc_ref)
            for kc in range(qt + 1):
                kr = pl.ds(kc * KC, KC)
                for h in range(H):
                    hs = slice(h * DH, (h + 1) * DH)
                    s = lax.dot_general(
                        qbf_ref[:, hs], kbf_ref[kr, hs],
                        (((1,), (1,)), ((), ())),
                        preferred_element_type=jnp.float32) * SCALE
                    if kc == qt:
                        i_blk = lax.broadcasted_iota(jnp.int32, (QT, KC), 0) // BLK
                        j_blk = lax.broadcasted_iota(jnp.int32, (QT, KC), 1) // BLK
                        w = jnp.where(j_blk <= i_blk, jnp.exp(s), 0.0)
                    else:
                        w = jnp.exp(s)
                    oacc_ref[:, hs] = oacc_ref[:, hs] + lax.dot_general(
                        w.astype(jnp.bfloat16), vbf_ref[kr, hs],
                        (((1,), (0,)), ((), ())),
                        preferred_element_type=jnp.float32)
                    lacc_ref[:, h:h + 1] = (lacc_ref[:, h:h + 1]
                                            + jnp.sum(w, axis=1, keepdims=True))
            for h in range(H):
                hs = slice(h * DH, (h + 1) * DH)
                ctx_ref[rows, hs] = (oacc_ref[:, hs]
                                     / lacc_ref[:, h:h + 1]
                                     ).astype(jnp.bfloat16)
            if qt < N_QT - 1:
                plan = (((qt,), ssemA, 1), ((qt,), ssemB, 3))
            else:
                plan = (((3, 4), ssemA, 1), ((4, 3), ssemB, 3))
            for slots, send_sems, target in plan:
                for slot in slots:
                    rdma = seg_rdma(slot, send_sems, slot, target)
                    rdma.start()
                    sends.append(rdma)
        for slot in range(N_SEM):
            project_out(slot)
        for rdma in sends:
            rdma.wait_send()

    @pl.when(my == 1)
    def _forwarder():
        fwds = []
        for slot in range(N_QT):
            seg_rdma(slot, ssemA, slot, 0).wait_recv()
            fwd = seg_rdma(slot, ssemB, slot, 2)
            fwd.start()
            fwds.append(fwd)
            project_out(slot)
        seg_rdma(4, ssemA, 4, 0).wait_recv()
        project_out(4)
        for fwd in fwds:
            fwd.wait_send()

    @pl.when(my == 3)
    def _forwarder_b():
        seg_rdma(4, ssemA, 4, 0).wait_recv()
        fwd = seg_rdma(4, ssemB, 4, 2)
        fwd.start()
        project_out(4)
        for slot in range(N_QT):
            seg_rdma(slot, ssemA, slot, 0).wait_recv()
            project_out(slot)
        fwd.wait_send()

    @pl.when(my == 2)
    def _receiver():
        for slot in range(N_SEM):
            seg_rdma(slot, ssemA, slot, 0).wait_recv()
            project_out(slot)

    @functools.partial(pl.run_scoped, exit_sem=pltpu.SemaphoreType.REGULAR)
    def _(exit_sem):
        for nbr in (left, right):
            pl.semaphore_signal(exit_sem, inc=1, device_id=(nbr,),
                                device_id_type=_MESH)
        pl.semaphore_wait(exit_sem, 2)


def kernel(x, Wq, K_ext, V_ext, Wo):
    x2 = x.reshape(SQ, D)
    k2 = K_ext.reshape(SKV_LOC, H * DH)
    v2 = V_ext.reshape(SKV_LOC, H * DH)
    out = pl.pallas_call(
        _body,
        out_shape=jax.ShapeDtypeStruct((SQ, D), jnp.float32),
        in_specs=[
            pl.BlockSpec(memory_space=pl.ANY),
            pl.BlockSpec(memory_space=pltpu.VMEM),
            pl.BlockSpec(memory_space=pltpu.VMEM),
            pl.BlockSpec(memory_space=pltpu.VMEM),
            pl.BlockSpec(memory_space=pltpu.VMEM),
        ],
        out_specs=pl.BlockSpec(memory_space=pltpu.VMEM),
        scratch_shapes=[
            pltpu.VMEM((SQ, D), jnp.bfloat16),
            pltpu.VMEM((2, QT, D), jnp.float32),
            pltpu.VMEM((QT, D), jnp.bfloat16),
            pltpu.VMEM((SKV_LOC, D), jnp.bfloat16),
            pltpu.VMEM((SKV_LOC, D), jnp.bfloat16),
            pltpu.VMEM((QT, D), jnp.float32),
            pltpu.VMEM((QT, 128), jnp.float32),
            pltpu.SemaphoreType.DMA((2,)),
            pltpu.SemaphoreType.DMA((N_SEM,)),
            pltpu.SemaphoreType.DMA((N_SEM,)),
            pltpu.SemaphoreType.DMA((N_SEM,)),
        ],
        compiler_params=pltpu.CompilerParams(
            collective_id=0,
            vmem_limit_bytes=100 * 1024 * 1024,
        ),
    )(x2, Wq, k2, v2, Wo)
    return out.reshape(1, SQ, D)


# device time: 96771 ns/iter; 1.5371x vs baseline; 1.0076x over previous
import functools

import jax
import jax.numpy as jnp
from jax import lax
from jax.experimental import pallas as pl
from jax.experimental.pallas import tpu as pltpu

N_DEV = 4
SQ = 2048
SKV_LOC = 2048
D = 1024
H = 8
DH = 128
QT = 512
N_QT = SQ // QT
KC = 512
BLK = 64
HT = QT // 2
SCALE = 0.08838834764831843
N_SEM = N_QT + 1

_MESH = pl.DeviceIdType.MESH


def _rows_for(slot):
    if slot < N_QT - 1:
        return pl.ds(slot * QT, QT)
    if slot == N_QT - 1:
        return pl.ds((N_QT - 1) * QT, HT)
    return pl.ds((N_QT - 1) * QT + HT, HT)


def _body(x_hbm_ref, wq_ref, k_ref, v_ref, wo_ref, out_ref,
          ctx_ref, xbuf_ref, qtile_ref, mask_ref,
          xsem, ssemA, ssemB, rsem):
    my = lax.axis_index("i")
    left = lax.rem(my + N_DEV - 1, N_DEV)
    right = lax.rem(my + 1, N_DEV)

    barrier = pltpu.get_barrier_semaphore()
    for nbr in (left, right):
        pl.semaphore_signal(barrier, inc=1, device_id=(nbr,),
                            device_id_type=_MESH)
    pl.semaphore_wait(barrier, 2)

    def seg_rdma(slot, send_sems, recv_slot, target):
        return pltpu.make_async_remote_copy(
            src_ref=ctx_ref.at[_rows_for(slot)],
            dst_ref=ctx_ref.at[_rows_for(recv_slot)],
            send_sem=send_sems.at[slot], recv_sem=rsem.at[recv_slot],
            device_id=(target,), device_id_type=_MESH)

    def project_out(slot):
        rows = _rows_for(slot)
        out_ref[rows, :] = jnp.dot(ctx_ref[rows, :].astype(jnp.float32),
                                   wo_ref[...],
                                   preferred_element_type=jnp.float32)

    def x_copy(qt, slot):
        return pltpu.make_async_copy(
            x_hbm_ref.at[pl.ds(qt * QT, QT)], xbuf_ref.at[slot],
            xsem.at[slot])

    @pl.when(my == 0)
    def _producer():
        i_blk = lax.broadcasted_iota(jnp.int32, (QT, KC), 0) // BLK
        j_blk = lax.broadcasted_iota(jnp.int32, (QT, KC), 1) // BLK
        mask_ref[...] = (j_blk <= i_blk).astype(jnp.float32)

        sends = []
        x_copy(0, 0).start()
        for qt in range(N_QT):
            rows = pl.ds(qt * QT, QT)
            x_copy(qt, qt % 2).wait()
            if qt + 1 < N_QT:
                x_copy(qt + 1, (qt + 1) % 2).start()
            qtile_ref[...] = jnp.dot(xbuf_ref[qt % 2], wq_ref[...],
                                     preferred_element_type=jnp.float32)
            kw = (qt + 1) * KC
            for h in range(H):
                hs = slice(h * DH, (h + 1) * DH)
                s = lax.dot_general(
                    qtile_ref[:, hs], k_ref[pl.ds(0, kw), hs],
                    (((1,), (1,)), ((), ())),
                    preferred_element_type=jnp.float32) * SCALE
                w = jnp.exp(s)
                wm = w[:, kw - KC:] * mask_ref[...]
                l = jnp.sum(wm, axis=1, keepdims=True)
                acc = lax.dot_general(
                    wm, v_ref[pl.ds(kw - KC, KC), hs],
                    (((1,), (0,)), ((), ())),
                    preferred_element_type=jnp.float32)
                if kw > KC:
                    wf = w[:, :kw - KC]
                    l = l + jnp.sum(wf, axis=1, keepdims=True)
                    acc = acc + lax.dot_general(
                        wf, v_ref[pl.ds(0, kw - KC), hs],
                        (((1,), (0,)), ((), ())),
                        preferred_element_type=jnp.float32)
                ctx_ref[rows, hs] = (acc / l).astype(jnp.bfloat16)
            if qt < N_QT - 1:
                plan = (((qt,), ssemA, 1), ((qt,), ssemB, 3))
            else:
                plan = (((3, 4), ssemA, 1), ((4, 3), ssemB, 3))
            for slots, send_sems, target in plan:
                for slot in slots:
                    rdma = seg_rdma(slot, send_sems, slot, target)
                    rdma.start()
                    sends.append(rdma)
        for slot in range(N_SEM):
            project_out(slot)
        for rdma in sends:
            rdma.wait_send()

    @pl.when(my == 1)
    def _forwarder():
        fwds = []
        for slot in range(N_QT):
            seg_rdma(slot, ssemA, slot, 0).wait_recv()
            fwd = seg_rdma(slot, ssemB, slot, 2)
            fwd.start()
            fwds.append(fwd)
            project_out(slot)
        seg_rdma(4, ssemA, 4, 0).wait_recv()
        project_out(4)
        for fwd in fwds:
            fwd.wait_send()

    @pl.when(my == 3)
    def _forwarder_b():
        seg_rdma(4, ssemA, 4, 0).wait_recv()
        fwd = seg_rdma(4, ssemB, 4, 2)
        fwd.start()
        project_out(4)
        for slot in range(N_QT):
            seg_rdma(slot, ssemA, slot, 0).wait_recv()
            project_out(slot)
        fwd.wait_send()

    @pl.when(my == 2)
    def _receiver():
        for slot in range(N_SEM):
            seg_rdma(slot, ssemA, slot, 0).wait_recv()
            project_out(slot)

    @functools.partial(pl.run_scoped, exit_sem=pltpu.SemaphoreType.REGULAR)
    def _(exit_sem):
        for nbr in (left, right):
            pl.semaphore_signal(exit_sem, inc=1, device_id=(nbr,),
                                device_id_type=_MESH)
        pl.semaphore_wait(exit_sem, 2)


def kernel(x, Wq, K_ext, V_ext, Wo):
    x2 = x.reshape(SQ, D)
    k2 = K_ext.reshape(SKV_LOC, H * DH)
    v2 = V_ext.reshape(SKV_LOC, H * DH)
    out = pl.pallas_call(
        _body,
        out_shape=jax.ShapeDtypeStruct((SQ, D), jnp.float32),
        in_specs=[
            pl.BlockSpec(memory_space=pl.ANY),
            pl.BlockSpec(memory_space=pltpu.VMEM),
            pl.BlockSpec(memory_space=pltpu.VMEM),
            pl.BlockSpec(memory_space=pltpu.VMEM),
            pl.BlockSpec(memory_space=pltpu.VMEM),
        ],
        out_specs=pl.BlockSpec(memory_space=pltpu.VMEM),
        scratch_shapes=[
            pltpu.VMEM((SQ, D), jnp.bfloat16),
            pltpu.VMEM((2, QT, D), jnp.float32),
            pltpu.VMEM((QT, D), jnp.float32),
            pltpu.VMEM((QT, KC), jnp.float32),
            pltpu.SemaphoreType.DMA((2,)),
            pltpu.SemaphoreType.DMA((N_SEM,)),
            pltpu.SemaphoreType.DMA((N_SEM,)),
            pltpu.SemaphoreType.DMA((N_SEM,)),
        ],
        compiler_params=pltpu.CompilerParams(
            collective_id=0,
            vmem_limit_bytes=100 * 1024 * 1024,
        ),
    )(x2, Wq, k2, v2, Wo)
    return out.reshape(1, SQ, D)
